# baseline (speedup 1.0000x reference)
import functools

import jax
import jax.numpy as jnp
from jax import lax
from jax.experimental import pallas as pl
from jax.experimental.pallas import tpu as pltpu
from jax.experimental.pallas import tpu_sc as plsc

N = 10000
E = 320000
D = 128
EPS = 1e-5

NC = 2
NS = 16
NW = NC * NS
CL = 128
NCHW = 80
IB = 20
NCHN = 80
NB = 4
E_PAD = NW * NCHW * CL
N_PAD = 10112
RPT = N_PAD // NS
N_PADN = 10240
RPTN = N_PADN // NS

_MESH = plsc.VectorSubcoreMesh(core_axis_name="c", subcore_axis_name="s")


def _sc_scatter_wide(h, ids, zblk):
    @functools.partial(
        pl.kernel,
        out_type=jax.ShapeDtypeStruct((NC, N_PAD, D), jnp.float32),
        mesh=_MESH,
        compiler_params=pltpu.CompilerParams(use_tc_tiling_on_sc=False),
        scratch_types=[
            pltpu.VMEM((2, IB, 2, CL), jnp.int32),
            [pltpu.VMEM((CL, D), jnp.float32)] * 2,
            [pltpu.SemaphoreType.DMA] * 2,
            [pltpu.SemaphoreType.DMA] * 2,
            [pltpu.SemaphoreType.DMA] * 2,
            pltpu.VMEM_SHARED((N_PAD, D), jnp.float32),
        ],
    )
    def k(h_hbm, ids_hbm, z_hbm, out_hbm, ids_v, bufs, isems, gsems,
          ssems, acc_sp):
        cid = lax.axis_index("c")
        sid = lax.axis_index("s")
        base = (cid * NS + sid) * NCHW
        for s in range(2):
            pltpu.async_copy(ids_hbm.at[pl.ds(base + s * IB, IB)],
                             ids_v.at[s], isems[s])
        r0 = sid * RPT
        pltpu.sync_copy(z_hbm, acc_sp.at[pl.ds(r0, RPT)])
        plsc.subcore_barrier()

        for bo in range(NCHW // IB):
            s = bo % 2
            pltpu.make_async_copy(
                ids_hbm.at[pl.ds(base + bo * IB, IB)], ids_v.at[s],
                isems[s]).wait()
            pltpu.async_copy(h_hbm.at[ids_v.at[s, 0, 0]], bufs[0],
                             gsems[0])

            def body(jo, carry):
                for bb in range(2):
                    jl = jo * 2 + bb
                    nb = 1 - bb
                    pltpu.make_async_copy(h_hbm.at[ids_v.at[s, jl, 0]],
                                          bufs[bb], gsems[bb]).wait()
                    pltpu.async_copy(bufs[bb],
                                     acc_sp.at[ids_v.at[s, jl, 1]],
                                     ssems[bb], add=True)

                    @pl.when(jl + 1 < IB)
                    def _():
                        @pl.when(jl >= 1)
                        def _():
                            pltpu.make_async_copy(
                                bufs[nb], acc_sp.at[ids_v.at[s, 0, 1]],
                                ssems[nb]).wait()

                        pltpu.async_copy(h_hbm.at[ids_v.at[s, jl + 1, 0]],
                                         bufs[nb], gsems[nb])
                return carry

            lax.fori_loop(0, IB // 2, body, 0)
            for bb in range(2):
                pltpu.make_async_copy(bufs[bb],
                                      acc_sp.at[ids_v.at[s, 0, 1]],
                                      ssems[bb]).wait()
            if bo + 2 < NCHW // IB:
                pltpu.async_copy(
                    ids_hbm.at[pl.ds(base + (bo + 2) * IB, IB)],
                    ids_v.at[s], isems[s])
        plsc.subcore_barrier()
        pltpu.sync_copy(acc_sp.at[pl.ds(r0, RPT)],
                        out_hbm.at[cid, pl.ds(r0, RPT)])

    return k(h, ids, zblk)


def _sc_deg(ids):
    @functools.partial(
        pl.kernel,
        out_type=jax.ShapeDtypeStruct((NC, N_PADN), jnp.float32),
        mesh=_MESH,
        scratch_types=[
            pltpu.VMEM((NCHN, 2, CL), jnp.int32),
            pltpu.VMEM((CL,), jnp.float32),
            pltpu.VMEM((RPTN,), jnp.float32),
            [pltpu.SemaphoreType.DMA] * NB,
            pltpu.VMEM_SHARED((N_PADN,), jnp.float32),
        ],
    )
    def k(ids_hbm, out_hbm, ids_v, ones_v, zbuf, ssems, acc_sp):
        cid = lax.axis_index("c")
        sid = lax.axis_index("s")
        c0 = (cid * NS + sid) * NCHN
        pltpu.sync_copy(ids_hbm.at[pl.ds(c0, NCHN)], ids_v)
        for j in range(CL // 16):
            ones_v[pl.ds(j * 16, 16)] = jnp.ones((16,), jnp.float32)
        for j in range(RPTN // 16):
            zbuf[pl.ds(j * 16, 16)] = jnp.zeros((16,), jnp.float32)
        r0 = sid * RPTN
        pltpu.sync_copy(zbuf, acc_sp.at[pl.ds(r0, RPTN)])
        plsc.subcore_barrier()

        for b in range(NB):
            pltpu.async_copy(ones_v, acc_sp.at[ids_v.at[b, 1]], ssems[b],
                             add=True)

        def body(jo, carry):
            for b in range(NB):
                j = jo * NB + b
                pltpu.make_async_copy(ones_v, acc_sp.at[ids_v.at[j, 1]],
                                      ssems[b]).wait()
                jn = j + NB

                @pl.when(jn < NCHN)
                def _():
                    pltpu.async_copy(ones_v, acc_sp.at[ids_v.at[jn, 1]],
                                     ssems[b], add=True)
            return carry

        lax.fori_loop(0, NCHN // NB, body, 0)
        plsc.subcore_barrier()
        pltpu.sync_copy(acc_sp.at[pl.ds(r0, RPTN)],
                        out_hbm.at[cid, pl.ds(r0, RPTN)])

    return k(ids)


def _sc_scatter_1d(vals, ids):
    @functools.partial(
        pl.kernel,
        out_type=jax.ShapeDtypeStruct((NC, N_PADN), jnp.float32),
        mesh=_MESH,
        scratch_types=[
            pltpu.VMEM((NCHN, 2, CL), jnp.int32),
            [pltpu.VMEM((CL,), jnp.float32)] * NB,
            [pltpu.SemaphoreType.DMA] * NB,
            [pltpu.SemaphoreType.DMA] * NB,
            pltpu.VMEM((RPTN,), jnp.float32),
            pltpu.VMEM_SHARED((N_PADN,), jnp.float32),
        ],
    )
    def k(v_hbm, ids_hbm, out_hbm, ids_v, bufs, gsems, ssems, zbuf,
          acc_sp):
        cid = lax.axis_index("c")
        sid = lax.axis_index("s")
        c0 = (cid * NS + sid) * NCHN
        pltpu.sync_copy(ids_hbm.at[pl.ds(c0, NCHN)], ids_v)
        for b in range(NB // 2):
            pltpu.async_copy(v_hbm.at[ids_v.at[b, 0]], bufs[b], gsems[b])
        for j in range(RPTN // 16):
            zbuf[pl.ds(j * 16, 16)] = jnp.zeros((16,), jnp.float32)
        r0 = sid * RPTN
        pltpu.sync_copy(zbuf, acc_sp.at[pl.ds(r0, RPTN)])
        plsc.subcore_barrier()

        def body(jo, carry):
            for bb in range(NB):
                j = jo * NB + bb
                jg = j + NB // 2
                bg = (bb + NB // 2) % NB

                @pl.when(jg < NCHN)
                def _():
                    @pl.when(j >= NB // 2)
                    def _():
                        pltpu.make_async_copy(bufs[bg],
                                              acc_sp.at[ids_v.at[0, 1]],
                                              ssems[bg]).wait()

                    pltpu.async_copy(v_hbm.at[ids_v.at[jg, 0]], bufs[bg],
                                     gsems[bg])

                pltpu.make_async_copy(v_hbm.at[ids_v.at[j, 0]], bufs[bb],
                                      gsems[bb]).wait()
                pltpu.async_copy(bufs[bb], acc_sp.at[ids_v.at[j, 1]],
                                 ssems[bb], add=True)
            return carry

        lax.fori_loop(0, NCHN // NB, body, 0)
        for bb in range(NB):
            pltpu.make_async_copy(bufs[bb], acc_sp.at[ids_v.at[0, 1]],
                                  ssems[bb]).wait()
        plsc.subcore_barrier()
        pltpu.sync_copy(acc_sp.at[pl.ds(r0, RPTN)],
                        out_hbm.at[cid, pl.ds(r0, RPTN)])

    return k(vals, ids)


def _tc_first(x, W0, degp):
    def body(x_ref, w_ref, degp_ref, d_ref, h_ref):
        deg = (degp_ref[0, :N] + degp_ref[1, :N] + 1.0).reshape(N, 1)
        d = lax.rsqrt(deg)
        d_ref[...] = d
        h_ref[...] = jnp.dot(x_ref[...], w_ref[...],
                             preferred_element_type=jnp.float32) * d

    return pl.pallas_call(
        body,
        out_shape=(jax.ShapeDtypeStruct((N, 1), jnp.float32),
                   jax.ShapeDtypeStruct((N, D), jnp.float32)),
    )(x, W0, degp)


def _tc_mid(acc, h, d, b, g, be, Wn, last):
    def body(acc_ref, h_ref, d_ref, b_ref, g_ref, be_ref, w_ref, o_ref):
        d_ = d_ref[...]
        z = acc_ref[0, :N, :] + acc_ref[1, :N, :] + h_ref[...]
        z = d_ * z + b_ref[...]
        mean = jnp.mean(z, axis=0, keepdims=True)
        zc = z - mean
        var = jnp.mean(zc * zc, axis=0, keepdims=True)
        y = g_ref[...] * zc * lax.rsqrt(var + EPS) + be_ref[...]
        y = jnp.maximum(y, 0.0)
        o_ref[...] = jnp.dot(y, w_ref[...],
                             preferred_element_type=jnp.float32) * d_

    out_sh = ((N, 1) if last else (N, D))
    return pl.pallas_call(
        body,
        out_shape=jax.ShapeDtypeStruct(out_sh, jnp.float32),
    )(acc, h, d, b.reshape(1, D), g.reshape(1, D), be.reshape(1, D), Wn)


def _tc_final(acc, h3, d, b3):
    def body(acc_ref, h3_ref, d_ref, b3_ref, o_ref):
        a = (acc_ref[0, :N] + acc_ref[1, :N]).reshape(N, 1)
        o_ref[...] = d_ref[...] * (a + h3_ref[...]) + b3_ref[...]

    return pl.pallas_call(
        body,
        out_shape=jax.ShapeDtypeStruct((N, 1), jnp.float32),
    )(acc, h3, d, b3.reshape(1, 1))


def kernel(x, edge_index, W0, b0, W1, b1, W2, b2, W3, b3,
           g0, be0, g1, be1, g2, be2):
    npad = E_PAD - E
    pad_src = jnp.zeros((npad,), jnp.int32)
    pad_dst = N + 16 + (jnp.arange(npad, dtype=jnp.int32) % (N_PAD - N - 16))
    src2d = jnp.concatenate([edge_index[0], pad_src]).reshape(-1, 1, CL)
    dst2d = jnp.concatenate([edge_index[1], pad_dst]).reshape(-1, 1, CL)
    ids = jnp.concatenate([src2d, dst2d], axis=1)
    zblk = jnp.zeros((RPT, D), jnp.float32)

    degp = _sc_deg(ids)
    d, h = _tc_first(x, W0, degp)

    acc = _sc_scatter_wide(h, ids, zblk)
    h = _tc_mid(acc, h, d, b0, g0, be0, W1, False)

    acc = _sc_scatter_wide(h, ids, zblk)
    h = _tc_mid(acc, h, d, b1, g1, be1, W2, False)

    acc = _sc_scatter_wide(h, ids, zblk)
    h3 = _tc_mid(acc, h, d, b2, g2, be2, W3, True)

    acc3 = _sc_scatter_1d(h3.reshape(-1), ids)
    out = _tc_final(acc3, h3, d, b3)
    return out.reshape(-1)


# speedup vs baseline: 2.6188x; 2.6188x over previous
import functools

import jax
import jax.numpy as jnp
from jax import lax
from jax.experimental import pallas as pl
from jax.experimental.pallas import tpu as pltpu
from jax.experimental.pallas import tpu_sc as plsc

N = 10000
E = 320000
D = 128
DH = D // 2
EPS = 1e-5

NC = 2
NS = 16
NW = NC * NS
CL = 128
NCHW = 160
IB = 20
NBW = 4
NCHN = 80
NB = 4
E_PAD = NS * NCHW * CL
N_TAB = 10240
TPT = N_TAB // NS
N_PAD = 10112
RPT = N_PAD // NS
N_PADN = 10240
RPTN = N_PADN // NS

_MESH = plsc.VectorSubcoreMesh(core_axis_name="c", subcore_axis_name="s")


def _sc_scatter_wide(h2, ids, zblk):
    @functools.partial(
        pl.kernel,
        out_type=jax.ShapeDtypeStruct((NC, N_PAD, DH), jnp.float32),
        mesh=_MESH,
        compiler_params=pltpu.CompilerParams(use_tc_tiling_on_sc=False),
        scratch_types=[
            pltpu.VMEM((2, IB, 2, CL), jnp.int32),
            [pltpu.VMEM((CL, DH), jnp.float32)] * NBW,
            [pltpu.SemaphoreType.DMA] * 2,
            [pltpu.SemaphoreType.DMA] * NBW,
            [pltpu.SemaphoreType.DMA] * NBW,
            pltpu.VMEM_SHARED((N_TAB, DH), jnp.float32),
            pltpu.VMEM_SHARED((N_PAD, DH), jnp.float32),
        ],
    )
    def k(h_hbm, ids_hbm, z_hbm, out_hbm, ids_v, bufs, isems, gsems,
          ssems, tab_sp, acc_sp):
        cid = lax.axis_index("c")
        sid = lax.axis_index("s")
        base = sid * NCHW
        for s in range(2):
            pltpu.async_copy(ids_hbm.at[pl.ds(base + s * IB, IB)],
                             ids_v.at[s], isems[s])
        t0 = sid * TPT
        pltpu.sync_copy(h_hbm.at[cid, pl.ds(t0, TPT)],
                        tab_sp.at[pl.ds(t0, TPT)])
        r0 = sid * RPT
        pltpu.sync_copy(z_hbm, acc_sp.at[pl.ds(r0, RPT)])
        plsc.subcore_barrier()

        for bo in range(NCHW // IB):
            s = bo % 2
            pltpu.make_async_copy(
                ids_hbm.at[pl.ds(base + bo * IB, IB)], ids_v.at[s],
                isems[s]).wait()
            for b in range(NBW // 2):
                pltpu.async_copy(tab_sp.at[ids_v.at[s, b, 0]], bufs[b],
                                 gsems[b])

            def body(jo, carry):
                for bb in range(NBW):
                    jl = jo * NBW + bb
                    jg = jl + NBW // 2
                    bg = (bb + NBW // 2) % NBW

                    @pl.when(jg < IB)
                    def _():
                        @pl.when(jl >= NBW // 2)
                        def _():
                            pltpu.make_async_copy(
                                bufs[bg], acc_sp.at[ids_v.at[s, 0, 1]],
                                ssems[bg]).wait()

                        pltpu.async_copy(
                            tab_sp.at[ids_v.at[s, jg, 0]], bufs[bg],
                            gsems[bg])

                    pltpu.make_async_copy(tab_sp.at[ids_v.at[s, jl, 0]],
                                          bufs[bb], gsems[bb]).wait()
                    pltpu.async_copy(bufs[bb],
                                     acc_sp.at[ids_v.at[s, jl, 1]],
                                     ssems[bb], add=True)
                return carry

            lax.fori_loop(0, IB // NBW, body, 0)
            for bb in range(NBW):
                pltpu.make_async_copy(bufs[bb],
                                      acc_sp.at[ids_v.at[s, 0, 1]],
                                      ssems[bb]).wait()
            if bo + 2 < NCHW // IB:
                pltpu.async_copy(
                    ids_hbm.at[pl.ds(base + (bo + 2) * IB, IB)],
                    ids_v.at[s], isems[s])
        plsc.subcore_barrier()
        pltpu.sync_copy(acc_sp.at[pl.ds(r0, RPT)],
                        out_hbm.at[cid, pl.ds(r0, RPT)])

    return k(h2, ids, zblk)


def _sc_deg(ids):
    @functools.partial(
        pl.kernel,
        out_type=jax.ShapeDtypeStruct((NC, N_PADN), jnp.float32),
        mesh=_MESH,
        scratch_types=[
            pltpu.VMEM((NCHN, 2, CL), jnp.int32),
            pltpu.VMEM((CL,), jnp.float32),
            pltpu.VMEM((RPTN,), jnp.float32),
            [pltpu.SemaphoreType.DMA] * NB,
            pltpu.VMEM_SHARED((N_PADN,), jnp.float32),
        ],
    )
    def k(ids_hbm, out_hbm, ids_v, ones_v, zbuf, ssems, acc_sp):
        cid = lax.axis_index("c")
        sid = lax.axis_index("s")
        c0 = (cid * NS + sid) * NCHN
        pltpu.sync_copy(ids_hbm.at[pl.ds(c0, NCHN)], ids_v)
        for j in range(CL // 16):
            ones_v[pl.ds(j * 16, 16)] = jnp.ones((16,), jnp.float32)
        for j in range(RPTN // 16):
            zbuf[pl.ds(j * 16, 16)] = jnp.zeros((16,), jnp.float32)
        r0 = sid * RPTN
        pltpu.sync_copy(zbuf, acc_sp.at[pl.ds(r0, RPTN)])
        plsc.subcore_barrier()

        for b in range(NB):
            pltpu.async_copy(ones_v, acc_sp.at[ids_v.at[b, 1]], ssems[b],
                             add=True)

        def body(jo, carry):
            for b in range(NB):
                j = jo * NB + b
                pltpu.make_async_copy(ones_v, acc_sp.at[ids_v.at[j, 1]],
                                      ssems[b]).wait()
                jn = j + NB

                @pl.when(jn < NCHN)
                def _():
                    pltpu.async_copy(ones_v, acc_sp.at[ids_v.at[jn, 1]],
                                     ssems[b], add=True)
            return carry

        lax.fori_loop(0, NCHN // NB, body, 0)
        plsc.subcore_barrier()
        pltpu.sync_copy(acc_sp.at[pl.ds(r0, RPTN)],
                        out_hbm.at[cid, pl.ds(r0, RPTN)])

    return k(ids)


def _sc_scatter_1d(vals, ids):
    @functools.partial(
        pl.kernel,
        out_type=jax.ShapeDtypeStruct((NC, N_PADN), jnp.float32),
        mesh=_MESH,
        scratch_types=[
            pltpu.VMEM((NCHN, 2, CL), jnp.int32),
            [pltpu.VMEM((CL,), jnp.float32)] * NB,
            [pltpu.SemaphoreType.DMA] * NB,
            [pltpu.SemaphoreType.DMA] * NB,
            pltpu.VMEM((RPTN,), jnp.float32),
            pltpu.VMEM_SHARED((N_PADN,), jnp.float32),
        ],
    )
    def k(v_hbm, ids_hbm, out_hbm, ids_v, bufs, gsems, ssems, zbuf,
          acc_sp):
        cid = lax.axis_index("c")
        sid = lax.axis_index("s")
        c0 = (cid * NS + sid) * NCHN
        pltpu.sync_copy(ids_hbm.at[pl.ds(c0, NCHN)], ids_v)
        for b in range(NB // 2):
            pltpu.async_copy(v_hbm.at[ids_v.at[b, 0]], bufs[b], gsems[b])
        for j in range(RPTN // 16):
            zbuf[pl.ds(j * 16, 16)] = jnp.zeros((16,), jnp.float32)
        r0 = sid * RPTN
        pltpu.sync_copy(zbuf, acc_sp.at[pl.ds(r0, RPTN)])
        plsc.subcore_barrier()

        def body(jo, carry):
            for bb in range(NB):
                j = jo * NB + bb
                jg = j + NB // 2
                bg = (bb + NB // 2) % NB

                @pl.when(jg < NCHN)
                def _():
                    @pl.when(j >= NB // 2)
                    def _():
                        pltpu.make_async_copy(bufs[bg],
                                              acc_sp.at[ids_v.at[0, 1]],
                                              ssems[bg]).wait()

                    pltpu.async_copy(v_hbm.at[ids_v.at[jg, 0]], bufs[bg],
                                     gsems[bg])

                pltpu.make_async_copy(v_hbm.at[ids_v.at[j, 0]], bufs[bb],
                                      gsems[bb]).wait()
                pltpu.async_copy(bufs[bb], acc_sp.at[ids_v.at[j, 1]],
                                 ssems[bb], add=True)
            return carry

        lax.fori_loop(0, NCHN // NB, body, 0)
        for bb in range(NB):
            pltpu.make_async_copy(bufs[bb], acc_sp.at[ids_v.at[0, 1]],
                                  ssems[bb]).wait()
        plsc.subcore_barrier()
        pltpu.sync_copy(acc_sp.at[pl.ds(r0, RPTN)],
                        out_hbm.at[cid, pl.ds(r0, RPTN)])

    return k(vals, ids)


def _tc_first(x, W0, degp):
    def body(x_ref, w_ref, degp_ref, d_ref, h_ref):
        deg = (degp_ref[0, :N] + degp_ref[1, :N] + 1.0).reshape(N, 1)
        d = lax.rsqrt(deg)
        d_ref[...] = d
        hp = jnp.dot(x_ref[...], w_ref[...],
                     preferred_element_type=jnp.float32) * d
        h_ref[0, :N, :] = hp[:, :DH]
        h_ref[1, :N, :] = hp[:, DH:]

    return pl.pallas_call(
        body,
        out_shape=(jax.ShapeDtypeStruct((N, 1), jnp.float32),
                   jax.ShapeDtypeStruct((2, N_TAB, DH), jnp.float32)),
    )(x, W0, degp)


def _tc_mid(acc, h2, d, b, g, be, Wn, last):
    def body(acc_ref, h_ref, d_ref, b_ref, g_ref, be_ref, w_ref, o_ref):
        d_ = d_ref[...]
        z = jnp.concatenate(
            [acc_ref[0, :N, :] + h_ref[0, :N, :],
             acc_ref[1, :N, :] + h_ref[1, :N, :]], axis=1)
        z = d_ * z + b_ref[...]
        mean = jnp.mean(z, axis=0, keepdims=True)
        zc = z - mean
        var = jnp.mean(zc * zc, axis=0, keepdims=True)
        y = g_ref[...] * zc * lax.rsqrt(var + EPS) + be_ref[...]
        y = jnp.maximum(y, 0.0)
        hp = jnp.dot(y, w_ref[...], preferred_element_type=jnp.float32) * d_
        if last:
            o_ref[...] = hp
        else:
            o_ref[0, :N, :] = hp[:, :DH]
            o_ref[1, :N, :] = hp[:, DH:]

    out_sh = ((N, 1) if last else (2, N_TAB, DH))
    return pl.pallas_call(
        body,
        out_shape=jax.ShapeDtypeStruct(out_sh, jnp.float32),
    )(acc, h2, d, b.reshape(1, D), g.reshape(1, D), be.reshape(1, D), Wn)


def _tc_final(acc, h3, d, b3):
    def body(acc_ref, h3_ref, d_ref, b3_ref, o_ref):
        a = (acc_ref[0, :N] + acc_ref[1, :N]).reshape(N, 1)
        o_ref[...] = d_ref[...] * (a + h3_ref[...]) + b3_ref[...]

    return pl.pallas_call(
        body,
        out_shape=jax.ShapeDtypeStruct((N, 1), jnp.float32),
    )(acc, h3, d, b3.reshape(1, 1))


def kernel(x, edge_index, W0, b0, W1, b1, W2, b2, W3, b3,
           g0, be0, g1, be1, g2, be2):
    npad = E_PAD - E
    pad_src = jnp.zeros((npad,), jnp.int32)
    pad_dst = N + 16 + (jnp.arange(npad, dtype=jnp.int32) % (N_PAD - N - 16))
    src2d = jnp.concatenate([edge_index[0], pad_src]).reshape(-1, 1, CL)
    dst2d = jnp.concatenate([edge_index[1], pad_dst]).reshape(-1, 1, CL)
    ids = jnp.concatenate([src2d, dst2d], axis=1)
    zblk = jnp.zeros((RPT, DH), jnp.float32)

    degp = _sc_deg(ids)
    d, h2 = _tc_first(x, W0, degp)

    acc = _sc_scatter_wide(h2, ids, zblk)
    h2 = _tc_mid(acc, h2, d, b0, g0, be0, W1, False)

    acc = _sc_scatter_wide(h2, ids, zblk)
    h2 = _tc_mid(acc, h2, d, b1, g1, be1, W2, False)

    acc = _sc_scatter_wide(h2, ids, zblk)
    h3 = _tc_mid(acc, h2, d, b2, g2, be2, W3, True)

    acc3 = _sc_scatter_1d(h3.reshape(-1), ids)
    out = _tc_final(acc3, h3, d, b3)
    return out.reshape(-1)
